# baseline (speedup 1.0000x reference)
import functools

import jax
import jax.numpy as jnp
from jax.experimental import pallas as pl
from jax.experimental.pallas import tpu as pltpu

B = 4
S = 2048
D = 1024
L = 10
E = 16
K = 2


def _tree_sum(x):
    n = x.shape[0]
    while n > 1:
        h = n // 2
        x = x[:h] + x[h:]
        n = h
    return x[0]


def _body(x_ref, w_ref, b_ref, p_ref, out_ref):
    parts = []
    for b in range(B):
        x = x_ref[b].reshape(S // 8, 8, D)
        parts.append(jnp.sum(_tree_sum(x), axis=0, keepdims=True))
    mean = jnp.concatenate(parts, axis=0) * (1.0 / S)
    logits = jax.lax.dot_general(
        mean, w_ref[...], (((1,), (1,)), ((), ())),
        preferred_element_type=jnp.float32) + b_ref[...]
    scores = jax.nn.softmax(logits, axis=-1)
    iota = jax.lax.broadcasted_iota(jnp.int32, (B, E), 1)
    big = jnp.int32(E)
    m1 = jnp.max(scores, axis=1, keepdims=True)
    i1 = jnp.min(jnp.where(scores == m1, iota, big), axis=1, keepdims=True)
    s2 = jnp.where(iota == i1, -jnp.inf, scores)
    m2 = jnp.max(s2, axis=1, keepdims=True)
    i2 = jnp.min(jnp.where(s2 == m2, iota, big), axis=1, keepdims=True)
    w1 = jnp.where(iota == i1, m1, 0.0)
    w2 = jnp.where(iota == i2, m2, 0.0)
    wmat = jnp.concatenate([w1[:, None, :], w2[:, None, :]], axis=1)
    wmat = wmat.reshape(2 * B, E)
    out_ref[...] = jax.lax.dot_general(
        wmat, p_ref[...], (((1,), (0,)), ((), ())),
        preferred_element_type=jnp.float32)


@jax.jit
def _run(x_embed, prompts, router_w, router_b):
    p2d = prompts.reshape(E, L * D)
    out2d = pl.pallas_call(
        _body,
        in_specs=[
            pl.BlockSpec((B, S, D), lambda: (0, 0, 0)),
            pl.BlockSpec((E, D), lambda: (0, 0)),
            pl.BlockSpec((1, E), lambda: (0, 0)),
            pl.BlockSpec((E, L * D), lambda: (0, 0)),
        ],
        out_specs=pl.BlockSpec((2 * B, L * D), lambda: (0, 0)),
        out_shape=jax.ShapeDtypeStruct((2 * B, L * D), jnp.float32),
    )(x_embed, router_w, router_b.reshape(1, E), p2d)
    return out2d.reshape(B, K * L, D)


def kernel(x_embed, prompts, router_w, router_b, layer_idx):
    return _run(x_embed, prompts, router_w, router_b)


# speedup vs baseline: 1.0962x; 1.0962x over previous
import functools

import jax
import jax.numpy as jnp
from jax.experimental import pallas as pl
from jax.experimental.pallas import tpu as pltpu

B = 4
S = 2048
D = 1024
L = 10
E = 16
K = 2
ROWS = 256
NCH = (B * S) // ROWS
NBUF = 4
CPB = S // ROWS


def _tree_sum(x):
    n = x.shape[0]
    while n > 1:
        h = n // 2
        x = x[:h] + x[h:]
        n = h
    return x[0]


def _body(x_ref, w_ref, b_ref, p_ref, out_ref,
          buf0, buf1, buf2, buf3, acc_ref, sems):
    bufs = (buf0, buf1, buf2, buf3)

    def start(c, j):
        pltpu.make_async_copy(
            x_ref.at[pl.ds(c * ROWS, ROWS), :], bufs[j], sems.at[j]
        ).start()

    def wait(c, j):
        pltpu.make_async_copy(
            x_ref.at[pl.ds(c * ROWS, ROWS), :], bufs[j], sems.at[j]
        ).wait()

    for j in range(NBUF):
        start(j, j)

    for b in range(B):
        def step(i, acc):
            for j in range(NBUF):
                c = b * CPB + i * NBUF + j
                wait(c, j)
                x = bufs[j][...].reshape(ROWS // 8, 8, D)
                acc = acc + _tree_sum(x)

                nxt = c + NBUF

                @pl.when(nxt < NCH)
                def _start():
                    start(nxt, j)
            return acc

        acc = jax.lax.fori_loop(
            0, CPB // NBUF, step, jnp.zeros((8, D), jnp.float32))
        acc_ref[pl.ds(b, 1), :] = jnp.sum(acc, axis=0, keepdims=True)

    mean = acc_ref[...] * (1.0 / S)
    logits = jax.lax.dot_general(
        mean, w_ref[...], (((1,), (1,)), ((), ())),
        preferred_element_type=jnp.float32) + b_ref[...]
    scores = jax.nn.softmax(logits, axis=-1)
    iota = jax.lax.broadcasted_iota(jnp.int32, (B, E), 1)
    big = jnp.int32(E)
    m1 = jnp.max(scores, axis=1, keepdims=True)
    i1 = jnp.min(jnp.where(scores == m1, iota, big), axis=1, keepdims=True)
    s2 = jnp.where(iota == i1, -jnp.inf, scores)
    m2 = jnp.max(s2, axis=1, keepdims=True)
    i2 = jnp.min(jnp.where(s2 == m2, iota, big), axis=1, keepdims=True)
    w1 = jnp.where(iota == i1, m1, 0.0)
    w2 = jnp.where(iota == i2, m2, 0.0)
    wmat = jnp.concatenate([w1[:, None, :], w2[:, None, :]], axis=1)
    wmat = wmat.reshape(2 * B, E)
    out_ref[...] = jax.lax.dot_general(
        wmat, p_ref[...], (((1,), (0,)), ((), ())),
        preferred_element_type=jnp.float32)


@jax.jit
def _run(x_embed, prompts, router_w, router_b):
    p2d = prompts.reshape(E, L * D)
    x2d = x_embed.reshape(B * S, D)
    out2d = pl.pallas_call(
        _body,
        in_specs=[
            pl.BlockSpec(memory_space=pltpu.MemorySpace.HBM),
            pl.BlockSpec((E, D), lambda: (0, 0)),
            pl.BlockSpec((1, E), lambda: (0, 0)),
            pl.BlockSpec((E, L * D), lambda: (0, 0)),
        ],
        out_specs=pl.BlockSpec((2 * B, L * D), lambda: (0, 0)),
        out_shape=jax.ShapeDtypeStruct((2 * B, L * D), jnp.float32),
        scratch_shapes=[
            pltpu.VMEM((ROWS, D), jnp.float32),
            pltpu.VMEM((ROWS, D), jnp.float32),
            pltpu.VMEM((ROWS, D), jnp.float32),
            pltpu.VMEM((ROWS, D), jnp.float32),
            pltpu.VMEM((B, D), jnp.float32),
            pltpu.SemaphoreType.DMA((NBUF,)),
        ],
    )(x2d, router_w, router_b.reshape(1, E), p2d)
    return out2d.reshape(B, K * L, D)


def kernel(x_embed, prompts, router_w, router_b, layer_idx):
    return _run(x_embed, prompts, router_w, router_b)


# speedup vs baseline: 1.2170x; 1.1102x over previous
import functools

import jax
import jax.numpy as jnp
from jax.experimental import pallas as pl
from jax.experimental.pallas import tpu as pltpu

B = 4
S = 2048
D = 1024
L = 10
E = 16
K = 2
CHUNK = 256
NSTEP = S // CHUNK


def _chunk_sum(x):
    n = x.shape[0]
    p = [x[0], x[1], x[2], x[3]]
    for g in range(1, n // 4):
        for k in range(4):
            p[k] = p[k] + x[4 * g + k]
    return (p[0] + p[1]) + (p[2] + p[3])


def _body(x_ref, w_ref, b_ref, p_ref, out_ref, acc_ref):
    i = pl.program_id(0)

    @pl.when(i == 0)
    def _init():
        acc_ref[...] = jnp.zeros_like(acc_ref)

    x = x_ref[...].reshape(B, CHUNK // 8, 8, D)
    for b in range(B):
        acc_ref[b] += _chunk_sum(x[b])

    @pl.when(i == NSTEP - 1)
    def _finish():
        mean = jnp.sum(acc_ref[...], axis=1) * (1.0 / S)
        logits = jax.lax.dot_general(
            mean, w_ref[...], (((1,), (1,)), ((), ())),
            preferred_element_type=jnp.float32) + b_ref[...]
        scores = jax.nn.softmax(logits, axis=-1)
        iota = jax.lax.broadcasted_iota(jnp.int32, (B, E), 1)
        big = jnp.int32(E)
        m1 = jnp.max(scores, axis=1, keepdims=True)
        i1 = jnp.min(jnp.where(scores == m1, iota, big), axis=1, keepdims=True)
        s2 = jnp.where(iota == i1, -jnp.inf, scores)
        m2 = jnp.max(s2, axis=1, keepdims=True)
        i2 = jnp.min(jnp.where(s2 == m2, iota, big), axis=1, keepdims=True)
        w1 = jnp.where(iota == i1, m1, 0.0)
        w2 = jnp.where(iota == i2, m2, 0.0)
        wmat = jnp.concatenate([w1[:, None, :], w2[:, None, :]], axis=1)
        wmat = wmat.reshape(2 * B, E)
        out_ref[...] = jax.lax.dot_general(
            wmat, p_ref[...], (((1,), (0,)), ((), ())),
            preferred_element_type=jnp.float32)


@jax.jit
def _run(x_embed, prompts, router_w, router_b):
    p2d = prompts.reshape(E, L * D)
    out2d = pl.pallas_call(
        _body,
        grid=(NSTEP,),
        in_specs=[
            pl.BlockSpec((B, CHUNK, D), lambda i: (0, i, 0)),
            pl.BlockSpec((E, D), lambda i: (0, 0)),
            pl.BlockSpec((1, E), lambda i: (0, 0)),
            pl.BlockSpec((E, L * D), lambda i: (0, 0)),
        ],
        out_specs=pl.BlockSpec((2 * B, L * D), lambda i: (0, 0)),
        out_shape=jax.ShapeDtypeStruct((2 * B, L * D), jnp.float32),
        scratch_shapes=[pltpu.VMEM((B, 8, D), jnp.float32)],
        compiler_params=pltpu.CompilerParams(
            dimension_semantics=("parallel",)),
    )(x_embed, router_w, router_b.reshape(1, E), p2d)
    return out2d.reshape(B, K * L, D)


def kernel(x_embed, prompts, router_w, router_b, layer_idx):
    return _run(x_embed, prompts, router_w, router_b)
